# baseline (speedup 1.0000x reference)
import functools
import jax
import jax.numpy as jnp
from jax import lax
from jax.experimental import pallas as pl
from jax.experimental.pallas import tpu as pltpu
from jax.experimental.pallas import tpu_sc as plsc

_N = 32768
_D = 1024
_N_SC = 8192
_N_TC = _N - _N_SC
_NC, _NS = 2, 16
_NW = _NC * _NS
_ROWS_W = _N_SC // _NW
_SCCHUNK = 32
_NSCCHUNK = _ROWS_W // _SCCHUNK

_CHUNK = 1024
_NBUF = 4
_NCHUNK = _N_TC // _CHUNK


def _tc_body(x_hbm, w_ref, b_ref, o_ref, bufs, sems):
    w = w_ref[...]
    bias = b_ref[0]
    for s in range(_NBUF):
        pltpu.make_async_copy(
            x_hbm.at[pl.ds(s * _CHUNK, _CHUNK), :], bufs.at[s], sems.at[s]
        ).start()

    def step(i, carry):
        slot = lax.rem(i, _NBUF)
        pltpu.make_async_copy(
            x_hbm.at[pl.ds(i * _CHUNK, _CHUNK), :], bufs.at[slot], sems.at[slot]
        ).wait()
        x = bufs[slot]
        logits = lax.dot_general(
            x, w, (((1,), (0,)), ((), ())),
            preferred_element_type=jnp.float32)
        o_ref[pl.ds(i * _CHUNK, _CHUNK)] = jnp.where(
            (logits[:, 0] + bias) > 0.0, 0, 1).astype(o_ref.dtype)
        nxt = i + _NBUF

        @pl.when(nxt < _NCHUNK)
        def _():
            pltpu.make_async_copy(
                x_hbm.at[pl.ds(nxt * _CHUNK, _CHUNK), :],
                bufs.at[slot], sems.at[slot]
            ).start()

        return carry

    lax.fori_loop(0, _NCHUNK, step, 0)


def _make_tc_call(out_dtype):
    return pl.pallas_call(
        _tc_body,
        in_specs=[
            pl.BlockSpec(memory_space=pltpu.HBM),
            pl.BlockSpec(memory_space=pltpu.VMEM),
            pl.BlockSpec(memory_space=pltpu.VMEM),
        ],
        out_specs=pl.BlockSpec(memory_space=pltpu.VMEM),
        out_shape=jax.ShapeDtypeStruct((_N_TC,), out_dtype),
        scratch_shapes=[
            pltpu.VMEM((_NBUF, _CHUNK, _D), jnp.float32),
            pltpu.SemaphoreType.DMA((_NBUF,)),
        ],
    )


def _sc_body(feat_hbm, w_hbm, b_hbm, out_hbm, xa, xb, wv, bv, outv, sema, semb):
    wid = lax.axis_index("s") * _NC + lax.axis_index("c")
    base = _N_TC + wid * _ROWS_W
    pltpu.sync_copy(w_hbm, wv)
    pltpu.sync_copy(b_hbm, bv)
    bias = bv[...][0]
    bufs = (xa, xb)
    sems = (sema, semb)
    lane = lax.iota(jnp.int32, 16)

    cps = []
    cps.append(pltpu.async_copy(
        feat_hbm.at[pl.ds(base, _SCCHUNK), :], xa, sema))
    for i in range(_NSCCHUNK):
        if i + 1 < _NSCCHUNK:
            cps.append(pltpu.async_copy(
                feat_hbm.at[pl.ds(base + (i + 1) * _SCCHUNK, _SCCHUNK), :],
                bufs[(i + 1) % 2], sems[(i + 1) % 2]))
        cps[i].wait()
        xbuf = bufs[i % 2]
        for g in range(_SCCHUNK // 16):
            def jbody(j, accs, xbuf=xbuf, g=g):
                wvec = wv[pl.ds(j * 16, 16)]
                return tuple(
                    accs[r] + xbuf[g * 16 + r, pl.ds(j * 16, 16)] * wvec
                    for r in range(16)
                )

            accs = lax.fori_loop(
                0, _D // 16, jbody,
                tuple(jnp.zeros((16,), jnp.float32) for _ in range(16)))
            dec = jnp.zeros((16,), jnp.int32)
            for r in range(16):
                s = jnp.sum(accs[r])
                d = jnp.where((s + bias) > 0.0, 0, 1)
                dec = jnp.where(lane == r, d, dec)
            outv[pl.ds((i * (_SCCHUNK // 16) + g) * 16, 16)] = \
                dec.astype(outv.dtype)
    pltpu.sync_copy(outv, out_hbm.at[pl.ds(wid * _ROWS_W, _ROWS_W)])


def _make_sc_call(out_dtype):
    return functools.partial(
        pl.kernel,
        out_type=jax.ShapeDtypeStruct((_N_SC,), out_dtype),
        mesh=plsc.VectorSubcoreMesh(core_axis_name="c", subcore_axis_name="s"),
        compiler_params=pltpu.CompilerParams(needs_layout_passes=False),
        scratch_types=[
            pltpu.VMEM((_SCCHUNK, _D), jnp.float32),
            pltpu.VMEM((_SCCHUNK, _D), jnp.float32),
            pltpu.VMEM((_D,), jnp.float32),
            pltpu.VMEM((16,), jnp.float32),
            pltpu.VMEM((_ROWS_W,), out_dtype),
            pltpu.SemaphoreType.DMA,
            pltpu.SemaphoreType.DMA,
        ],
    )(_sc_body)


def kernel(feat0, feat1, feat2, w, b):
    del feat0, feat2
    out_dtype = jnp.zeros((), dtype=jnp.int64).dtype
    w2 = w.reshape(_D, 1)
    b1 = b.reshape(1)
    b16 = jnp.full((16,), b, jnp.float32)
    sc_out = _make_sc_call(out_dtype)(feat1, w, b16)
    xla_dec = jnp.where((feat1 @ w + b) > 0.0,
                        jnp.asarray(0, out_dtype), jnp.asarray(1, out_dtype))
    return jnp.concatenate([xla_dec[:_N_TC], sc_out])


# speedup vs baseline: 1.1557x; 1.1557x over previous
import jax
import jax.numpy as jnp
from jax import lax
from jax.experimental import pallas as pl
from jax.experimental.pallas import tpu as pltpu

_N = 32768
_D = 1024
_CHUNK = 1024
_NBUF = 4
_NCHUNK = _N // _CHUNK


def _innernode_tc_kernel(x_hbm, w_ref, b_ref, o_ref, bufs, sems):
    w = w_ref[...]
    bias = b_ref[0]

    for s in range(_NBUF):
        pltpu.make_async_copy(
            x_hbm.at[pl.ds(s * _CHUNK, _CHUNK), :], bufs.at[s], sems.at[s]
        ).start()

    for i in range(_NCHUNK):
        slot = i % _NBUF
        pltpu.make_async_copy(
            x_hbm.at[pl.ds(i * _CHUNK, _CHUNK), :], bufs.at[slot], sems.at[slot]
        ).wait()
        x = bufs[slot]
        logits = lax.dot_general(
            x, w, (((1,), (0,)), ((), ())),
            preferred_element_type=jnp.float32)
        o_ref[pl.ds(i * _CHUNK, _CHUNK), :] = jnp.where(
            (logits + bias) > 0.0, 0, 1).astype(o_ref.dtype)
        nxt = i + _NBUF
        if nxt < _NCHUNK:
            pltpu.make_async_copy(
                x_hbm.at[pl.ds(nxt * _CHUNK, _CHUNK), :],
                bufs.at[slot], sems.at[slot]
            ).start()


def kernel(feat0, feat1, feat2, w, b):
    del feat0, feat2
    N, d = feat1.shape
    out_dtype = jnp.zeros((), dtype=jnp.int64).dtype
    w2 = w.reshape(d, 1)
    b1 = b.reshape(1)
    out2d = pl.pallas_call(
        _innernode_tc_kernel,
        in_specs=[
            pl.BlockSpec(memory_space=pltpu.HBM),
            pl.BlockSpec(memory_space=pltpu.VMEM),
            pl.BlockSpec(memory_space=pltpu.VMEM),
        ],
        out_specs=pl.BlockSpec(memory_space=pltpu.VMEM),
        out_shape=jax.ShapeDtypeStruct((N, 1), out_dtype),
        scratch_shapes=[
            pltpu.VMEM((_NBUF, _CHUNK, d), jnp.float32),
            pltpu.SemaphoreType.DMA((_NBUF,)),
        ],
    )(feat1, w2, b1)
    return out2d.reshape(N)


# speedup vs baseline: 1.5259x; 1.3203x over previous
import jax
import jax.numpy as jnp
from jax import lax
from jax.experimental import pallas as pl
from jax.experimental.pallas import tpu as pltpu

_N = 32768
_D = 1024
_CHUNK = 1024
_NBUF = 4
_NCHUNK = _N // _CHUNK


def _innernode_tc_kernel(x_hbm, w_ref, b_ref, o_ref, bufs, sems):
    w = w_ref[...]
    bias = b_ref[0]

    for s in range(_NBUF):
        pltpu.make_async_copy(
            x_hbm.at[pl.ds(s * _CHUNK, _CHUNK), :], bufs.at[s], sems.at[s]
        ).start()

    for i in range(_NCHUNK):
        slot = i % _NBUF
        pltpu.make_async_copy(
            x_hbm.at[pl.ds(i * _CHUNK, _CHUNK), :], bufs.at[slot], sems.at[slot]
        ).wait()
        x = bufs[slot]
        logits = lax.dot_general(
            x, w, (((1,), (0,)), ((), ())),
            preferred_element_type=jnp.float32)
        o_ref[pl.ds(i * _CHUNK, _CHUNK)] = jnp.where(
            (logits[:, 0] + bias) > 0.0, 0, 1).astype(o_ref.dtype)
        nxt = i + _NBUF
        if nxt < _NCHUNK:
            pltpu.make_async_copy(
                x_hbm.at[pl.ds(nxt * _CHUNK, _CHUNK), :],
                bufs.at[slot], sems.at[slot]
            ).start()


def kernel(feat0, feat1, feat2, w, b):
    del feat0, feat2
    N, d = feat1.shape
    out_dtype = jnp.zeros((), dtype=jnp.int64).dtype
    w2 = w.reshape(d, 1)
    b1 = b.reshape(1)
    return pl.pallas_call(
        _innernode_tc_kernel,
        in_specs=[
            pl.BlockSpec(memory_space=pltpu.HBM),
            pl.BlockSpec(memory_space=pltpu.VMEM),
            pl.BlockSpec(memory_space=pltpu.VMEM),
        ],
        out_specs=pl.BlockSpec(memory_space=pltpu.VMEM),
        out_shape=jax.ShapeDtypeStruct((N,), out_dtype),
        scratch_shapes=[
            pltpu.VMEM((_NBUF, _CHUNK, d), jnp.float32),
            pltpu.SemaphoreType.DMA((_NBUF,)),
        ],
    )(feat1, w2, b1)
